# Initial kernel scaffold; baseline (speedup 1.0000x reference)
import jax
import jax.numpy as jnp
from jax.experimental import pallas as pl


def kernel(coord, span, lc_coords, lc_span, ct_ind, ct_mask):
    raise NotImplementedError("write your pallas kernel here")


# speedup vs baseline: 2.1161x; 2.1161x over previous
import functools

import jax
import jax.numpy as jnp
from jax import lax
from jax.experimental import pallas as pl
from jax.experimental.pallas import tpu as pltpu
from jax.experimental.pallas import tpu_sc as plsc

_EPS = 1e-4

_B, _N, _H, _W = 16, 1024, 128, 128
_HW = _H * _W
_NC, _NS, _L = 2, 16, 16
_NW = _NC * _NS
_KC = (_N * 4) // 2
_KS = _N // 2

_mesh = plsc.VectorSubcoreMesh(core_axis_name="c", subcore_axis_name="s")


@functools.partial(
    pl.kernel,
    mesh=_mesh,
    out_type=jax.ShapeDtypeStruct((_NW * 48,), jnp.float32),
    compiler_params=pltpu.CompilerParams(needs_layout_passes=False),
    scratch_types=[
        pltpu.VMEM((2 * _HW,), jnp.float32),
        pltpu.VMEM((2 * _HW,), jnp.float32),
        pltpu.VMEM((_KC * 3,), jnp.int32),
        pltpu.VMEM((_KS * 2,), jnp.float32),
        pltpu.VMEM((_KS,), jnp.int32),
        pltpu.VMEM((_KS,), jnp.float32),
        pltpu.VMEM((48,), jnp.float32),
    ],
)
def _sc_loss(coord_hbm, span_hbm, lc_hbm, lcs_hbm, ci_hbm, cm_hbm, out_hbm,
             coord_v, span_v, lc_v, lcs_v, ci_v, cm_v, stage_v):
    b = lax.axis_index("s")
    h = lax.axis_index("c")
    wid = b * _NC + h

    pltpu.sync_copy(coord_hbm.at[pl.ds(b * 2 * _HW, 2 * _HW)], coord_v)
    pltpu.sync_copy(span_hbm.at[pl.ds(b * 2 * _HW, 2 * _HW)], span_v)
    pltpu.sync_copy(lc_hbm.at[pl.ds(b * (_N * 4 * 3) + h * (_KC * 3), _KC * 3)], lc_v)
    pltpu.sync_copy(lcs_hbm.at[pl.ds(b * (_N * 2) + h * (_KS * 2), _KS * 2)], lcs_v)
    pltpu.sync_copy(ci_hbm.at[pl.ds(b * _N + h * _KS, _KS)], ci_v)
    pltpu.sync_copy(cm_hbm.at[pl.ds(b * _N + h * _KS, _KS)], cm_v)

    iota = lax.iota(jnp.int32, _L)

    def coord_body(i, acc):
        pos = iota + i * _L
        p3 = pos * 3
        idx = plsc.load_gather(lc_v, [p3])
        colgt = plsc.load_gather(lc_v, [p3 + 1]).astype(jnp.float32)
        rowgt = plsc.load_gather(lc_v, [p3 + 2]).astype(jnp.float32)
        colp = plsc.load_gather(coord_v, [idx])
        rowp = plsc.load_gather(coord_v, [idx + _HW])
        m = plsc.load_gather(cm_v, [lax.shift_right_logical(pos, 2)])
        d = jnp.abs(colp * m - colgt * m) + jnp.abs(rowp * m - rowgt * m)
        return acc + d

    acc_c = lax.fori_loop(0, _KC // _L, coord_body, jnp.zeros((_L,), jnp.float32))

    def span_body(i, carry):
        acc_s, acc_m = carry
        pos = iota + i * _L
        ind = plsc.load_gather(ci_v, [pos])
        sp0 = plsc.load_gather(span_v, [ind])
        sp1 = plsc.load_gather(span_v, [ind + _HW])
        g0 = plsc.load_gather(lcs_v, [pos * 2])
        g1 = plsc.load_gather(lcs_v, [pos * 2 + 1])
        m = plsc.load_gather(cm_v, [pos])
        d = jnp.abs(sp0 * m - g0 * m) + jnp.abs(sp1 * m - g1 * m)
        return acc_s + d, acc_m + m

    acc_s, acc_m = lax.fori_loop(
        0, _KS // _L, span_body,
        (jnp.zeros((_L,), jnp.float32), jnp.zeros((_L,), jnp.float32)))

    stage_v[pl.ds(0, _L)] = acc_c
    stage_v[pl.ds(_L, _L)] = acc_s
    stage_v[pl.ds(2 * _L, _L)] = acc_m
    pltpu.sync_copy(stage_v, out_hbm.at[pl.ds(wid * 48, 48)])


def kernel(coord, span, lc_coords, lc_span, ct_ind, ct_mask):
    partials = _sc_loss(
        coord.reshape(_B * 2 * _HW),
        span.reshape(_B * 2 * _HW),
        lc_coords.reshape(_B * _N * 4 * 3),
        lc_span.reshape(_B * _N * 2),
        ct_ind.reshape(_B * _N),
        ct_mask.reshape(_B * _N),
    )
    sums = partials.reshape(_NW, 3, _L).sum(axis=(0, 2))
    mask_sum = sums[2]
    coord_loss = sums[0] / (4.0 * mask_sum + _EPS)
    span_loss = sums[1] / (2.0 * mask_sum + _EPS)
    return (coord_loss, jnp.array(0.0, dtype=jnp.float32), span_loss)


# speedup vs baseline: 5.7805x; 2.7316x over previous
import functools

import jax
import jax.numpy as jnp
from jax import lax
from jax.experimental import pallas as pl
from jax.experimental.pallas import tpu as pltpu
from jax.experimental.pallas import tpu_sc as plsc

_EPS = 1e-4

_B, _N, _H, _W = 16, 1024, 128, 128
_HW = _H * _W
_NC, _NS, _L = 2, 16, 16
_NW = _NC * _NS
_KC = _N * 4

_mesh = plsc.VectorSubcoreMesh(core_axis_name="c", subcore_axis_name="s")


@functools.partial(
    pl.kernel,
    mesh=_mesh,
    out_type=jax.ShapeDtypeStruct((_NW * 48,), jnp.float32),
    compiler_params=pltpu.CompilerParams(needs_layout_passes=False),
    scratch_types=[
        pltpu.VMEM((_HW,), jnp.float32),
        pltpu.VMEM((_HW,), jnp.float32),
        pltpu.VMEM((_KC,), jnp.int32),
        pltpu.VMEM((_KC,), jnp.int32),
        pltpu.VMEM((_N,), jnp.int32),
        pltpu.VMEM((_N,), jnp.float32),
        pltpu.VMEM((_N,), jnp.float32),
        pltpu.VMEM((48,), jnp.float32),
    ],
)
def _sc_loss(coord_hbm, span_hbm, lc_hbm, ci_hbm, lcs_hbm, cm_hbm,
             out_hbm, coord_v, span_v, idx_v, gt_v, ci_v, lcs_v, cm_v, stage_v):
    b = lax.axis_index("s")
    c = lax.axis_index("c")
    wid = b * _NC + c

    pltpu.sync_copy(coord_hbm.at[pl.ds((b * 2 + c) * _HW, _HW)], coord_v)
    pltpu.sync_copy(span_hbm.at[pl.ds((b * 2 + c) * _HW, _HW)], span_v)
    pltpu.sync_copy(lc_hbm.at[pl.ds(b * 3 * _KC, _KC)], idx_v)
    pltpu.sync_copy(lc_hbm.at[pl.ds(b * 3 * _KC + (1 + c) * _KC, _KC)], gt_v)
    pltpu.sync_copy(ci_hbm.at[pl.ds(b * _N, _N)], ci_v)
    pltpu.sync_copy(lcs_hbm.at[pl.ds((b * 2 + c) * _N, _N)], lcs_v)
    pltpu.sync_copy(cm_hbm.at[pl.ds(b * _N, _N)], cm_v)

    def coord_body(g, acc):
        p0 = g * _L
        idx = idx_v[pl.ds(p0, _L)]
        gt = gt_v[pl.ds(p0, _L)].astype(jnp.float32)
        pred = plsc.load_gather(coord_v, [idx])
        m = cm_v[pl.ds((g & (_N // _L - 1)) * _L, _L)]
        return acc + jnp.abs(pred * m - gt * m)

    acc_c = lax.fori_loop(0, _KC // _L, coord_body, jnp.zeros((_L,), jnp.float32))

    msel = jnp.where(c == 0, 1.0, 0.0).astype(jnp.float32)

    def span_body(g, carry):
        acc_s, acc_m = carry
        p0 = g * _L
        ind = ci_v[pl.ds(p0, _L)]
        sp = plsc.load_gather(span_v, [ind])
        gt = lcs_v[pl.ds(p0, _L)]
        m = cm_v[pl.ds(p0, _L)]
        return acc_s + jnp.abs(sp * m - gt * m), acc_m + m * msel

    acc_s, acc_m = lax.fori_loop(
        0, _N // _L, span_body,
        (jnp.zeros((_L,), jnp.float32), jnp.zeros((_L,), jnp.float32)))

    stage_v[pl.ds(0, _L)] = acc_c
    stage_v[pl.ds(_L, _L)] = acc_s
    stage_v[pl.ds(2 * _L, _L)] = acc_m
    pltpu.sync_copy(stage_v, out_hbm.at[pl.ds(wid * 48, 48)])


def kernel(coord, span, lc_coords, lc_span, ct_ind, ct_mask):
    lct = jnp.transpose(lc_coords, (0, 3, 2, 1))
    lst = jnp.transpose(lc_span, (0, 2, 1))
    partials = _sc_loss(
        coord.reshape(_B * 2 * _HW),
        span.reshape(_B * 2 * _HW),
        lct.reshape(_B * 3 * _KC),
        ct_ind.reshape(_B * _N),
        lst.reshape(_B * 2 * _N),
        ct_mask.reshape(_B * _N),
    )
    sums = partials.reshape(_NW, 3, _L).sum(axis=(0, 2))
    mask_sum = sums[2]
    coord_loss = sums[0] / (4.0 * mask_sum + _EPS)
    span_loss = sums[1] / (2.0 * mask_sum + _EPS)
    return (coord_loss, jnp.array(0.0, dtype=jnp.float32), span_loss)


# speedup vs baseline: 7.0760x; 1.2241x over previous
import functools

import jax
import jax.numpy as jnp
from jax import lax
from jax.experimental import pallas as pl
from jax.experimental.pallas import tpu as pltpu
from jax.experimental.pallas import tpu_sc as plsc

_EPS = 1e-4

_B, _N, _H, _W = 16, 1024, 128, 128
_HW = _H * _W
_NC, _NS, _L = 2, 16, 16
_NW = _NC * _NS
_KC = _N * 4

_mesh = plsc.VectorSubcoreMesh(core_axis_name="c", subcore_axis_name="s")


@functools.partial(
    pl.kernel,
    mesh=_mesh,
    out_type=jax.ShapeDtypeStruct((_NW * 48,), jnp.float32),
    compiler_params=pltpu.CompilerParams(needs_layout_passes=False),
    scratch_types=[
        pltpu.VMEM((_HW,), jnp.float32),
        pltpu.VMEM((_HW,), jnp.float32),
        pltpu.VMEM((_KC,), jnp.int32),
        pltpu.VMEM((_KC,), jnp.int32),
        pltpu.VMEM((_N,), jnp.int32),
        pltpu.VMEM((2 * _N,), jnp.float32),
        pltpu.VMEM((_N,), jnp.float32),
        pltpu.VMEM((48,), jnp.float32),
        pltpu.SemaphoreType.DMA,
    ],
)
def _sc_loss(coord_hbm, span_hbm, lc_hbm, ci_hbm, lcs_hbm, cm_hbm,
             out_hbm, coord_v, span_v, idx_v, gt_v, ci_v, lcs_v, cm_v,
             stage_v, sem):
    b = lax.axis_index("s")
    c = lax.axis_index("c")
    wid = b * _NC + c
    bt = lax.shift_right_logical(b, 3)
    bs = b & 7

    cps = []
    cps.append(pltpu.async_copy(
        coord_hbm.at[pl.ds((b * 2 + c) * _HW, _HW)], coord_v, sem))
    cps.append(pltpu.async_copy(
        span_hbm.at[pl.ds((b * 2 + c) * _HW, _HW)], span_v, sem))
    cps.append(pltpu.async_copy(
        lc_hbm.at[pl.ds(b * 3 * _KC, _KC)], idx_v, sem))
    cps.append(pltpu.async_copy(
        lc_hbm.at[pl.ds((b * 3 + 1 + c) * _KC, _KC)], gt_v, sem))
    cps.append(pltpu.async_copy(
        lcs_hbm.at[pl.ds(b * 2 * _N, 2 * _N)], lcs_v, sem))
    for nt in range(8):
        src = bt * 8192 + nt * 1024 + bs * 128
        cps.append(pltpu.async_copy(
            ci_hbm.at[pl.ds(src, 128)], ci_v.at[pl.ds(nt * 128, 128)], sem))
        cps.append(pltpu.async_copy(
            cm_hbm.at[pl.ds(src, 128)], cm_v.at[pl.ds(nt * 128, 128)], sem))
    for cp in cps:
        cp.wait()

    def coord_body(g, acc):
        p0 = g * _L
        idx = idx_v[pl.ds(p0, _L)]
        gt = gt_v[pl.ds(p0, _L)].astype(jnp.float32)
        pred = plsc.load_gather(coord_v, [idx])
        m0 = lax.shift_right_logical(g, 5) * 128 + (g & 7) * _L
        m = cm_v[pl.ds(m0, _L)]
        return acc + jnp.abs(pred * m - gt * m)

    acc_c = lax.fori_loop(0, _KC // _L, coord_body, jnp.zeros((_L,), jnp.float32))

    msel = jnp.where(c == 0, 1.0, 0.0).astype(jnp.float32)

    def span_body(g, carry):
        acc_s, acc_m = carry
        p0 = g * _L
        ind = ci_v[pl.ds(p0, _L)]
        sp = plsc.load_gather(span_v, [ind])
        g0 = lax.shift_right_logical(g, 3) * 256 + c * 128 + (g & 7) * _L
        gt = lcs_v[pl.ds(g0, _L)]
        m = cm_v[pl.ds(p0, _L)]
        return acc_s + jnp.abs(sp * m - gt * m), acc_m + m * msel

    acc_s, acc_m = lax.fori_loop(
        0, _N // _L, span_body,
        (jnp.zeros((_L,), jnp.float32), jnp.zeros((_L,), jnp.float32)))

    stage_v[pl.ds(0, _L)] = acc_c
    stage_v[pl.ds(_L, _L)] = acc_s
    stage_v[pl.ds(2 * _L, _L)] = acc_m
    pltpu.sync_copy(stage_v, out_hbm.at[pl.ds(wid * 48, 48)])


def kernel(coord, span, lc_coords, lc_span, ct_ind, ct_mask):
    lc_lin = (lc_coords.reshape(_B, 8, 128, 4, 3)
              .transpose(0, 4, 1, 3, 2)
              .reshape(_B * 3 * _KC))
    lcs_lin = (lc_span.reshape(_B, 8, 128, 2)
               .transpose(0, 1, 3, 2)
               .reshape(_B * 2 * _N))
    ci_lin = (ct_ind.reshape(2, 8, 8, 128)
              .transpose(0, 2, 1, 3)
              .reshape(_B * _N))
    cm_lin = (ct_mask.reshape(2, 8, 8, 128)
              .transpose(0, 2, 1, 3)
              .reshape(_B * _N))
    partials = _sc_loss(
        coord.reshape(_B * 2 * _HW),
        span.reshape(_B * 2 * _HW),
        lc_lin,
        ci_lin,
        lcs_lin,
        cm_lin,
    )
    sums = partials.reshape(_NW, 3, _L).sum(axis=(0, 2))
    mask_sum = sums[2]
    coord_loss = sums[0] / (4.0 * mask_sum + _EPS)
    span_loss = sums[1] / (2.0 * mask_sum + _EPS)
    return (coord_loss, jnp.array(0.0, dtype=jnp.float32), span_loss)


# speedup vs baseline: 7.1272x; 1.0072x over previous
import functools

import jax
import jax.numpy as jnp
from jax import lax
from jax.experimental import pallas as pl
from jax.experimental.pallas import tpu as pltpu
from jax.experimental.pallas import tpu_sc as plsc

_EPS = 1e-4

_B, _N, _H, _W = 16, 1024, 128, 128
_HW = _H * _W
_NC, _NS, _L = 2, 16, 16
_NW = _NC * _NS
_KC = _N * 4

_mesh = plsc.VectorSubcoreMesh(core_axis_name="c", subcore_axis_name="s")


@functools.partial(
    pl.kernel,
    mesh=_mesh,
    out_type=jax.ShapeDtypeStruct((_NW * 48,), jnp.float32),
    compiler_params=pltpu.CompilerParams(needs_layout_passes=False),
    scratch_types=[
        pltpu.VMEM((_HW,), jnp.float32),
        pltpu.VMEM((_HW,), jnp.float32),
        pltpu.VMEM((_KC,), jnp.int32),
        pltpu.VMEM((_KC,), jnp.int32),
        pltpu.VMEM((8, 128), jnp.int32),
        pltpu.VMEM((2 * _N,), jnp.float32),
        pltpu.VMEM((8, 128), jnp.float32),
        pltpu.VMEM((48,), jnp.float32),
        pltpu.SemaphoreType.DMA,
    ],
)
def _sc_loss(coord_hbm, span_hbm, lc_hbm, ci_hbm, lcs_hbm, cm_hbm,
             out_hbm, coord_v, span_v, idx_v, gt_v, ci_v, lcs_v, cm_v,
             stage_v, sem):
    b = lax.axis_index("s")
    c = lax.axis_index("c")
    wid = b * _NC + c
    bt = lax.shift_right_logical(b, 3)
    bs = b & 7

    cps = []
    cps.append(pltpu.async_copy(
        coord_hbm.at[pl.ds((b * 2 + c) * _HW, _HW)], coord_v, sem))
    cps.append(pltpu.async_copy(
        span_hbm.at[pl.ds((b * 2 + c) * _HW, _HW)], span_v, sem))
    cps.append(pltpu.async_copy(
        lc_hbm.at[pl.ds(b * 3 * _KC, _KC)], idx_v, sem))
    cps.append(pltpu.async_copy(
        lc_hbm.at[pl.ds((b * 3 + 1 + c) * _KC, _KC)], gt_v, sem))
    cps.append(pltpu.async_copy(
        lcs_hbm.at[pl.ds(b * 2 * _N, 2 * _N)], lcs_v, sem))
    cps.append(pltpu.async_copy(ci_hbm.at[bt, :, bs], ci_v, sem))
    cps.append(pltpu.async_copy(cm_hbm.at[bt, :, bs], cm_v, sem))
    for cp in cps:
        cp.wait()

    def coord_body(g, acc):
        p0 = g * _L
        idx = idx_v[pl.ds(p0, _L)]
        gt = gt_v[pl.ds(p0, _L)].astype(jnp.float32)
        pred = plsc.load_gather(coord_v, [idx])
        m = cm_v[lax.shift_right_logical(g, 5), pl.ds((g & 7) * _L, _L)]
        return acc + jnp.abs(pred * m - gt * m)

    acc_c = lax.fori_loop(0, _KC // _L, coord_body, jnp.zeros((_L,), jnp.float32))

    msel = jnp.where(c == 0, 1.0, 0.0).astype(jnp.float32)

    def span_body(g, carry):
        acc_s, acc_m = carry
        nt = lax.shift_right_logical(g, 3)
        nl0 = (g & 7) * _L
        ind = ci_v[nt, pl.ds(nl0, _L)]
        sp = plsc.load_gather(span_v, [ind])
        gt = lcs_v[pl.ds(nt * 256 + c * 128 + nl0, _L)]
        m = cm_v[nt, pl.ds(nl0, _L)]
        return acc_s + jnp.abs(sp * m - gt * m), acc_m + m * msel

    acc_s, acc_m = lax.fori_loop(
        0, _N // _L, span_body,
        (jnp.zeros((_L,), jnp.float32), jnp.zeros((_L,), jnp.float32)))

    stage_v[pl.ds(0, _L)] = acc_c
    stage_v[pl.ds(_L, _L)] = acc_s
    stage_v[pl.ds(2 * _L, _L)] = acc_m
    pltpu.sync_copy(stage_v, out_hbm.at[pl.ds(wid * 48, 48)])


def kernel(coord, span, lc_coords, lc_span, ct_ind, ct_mask):
    lc_lin = (lc_coords.reshape(_B, 8, 128, 4, 3)
              .transpose(0, 4, 1, 3, 2)
              .reshape(_B * 3 * _KC))
    lcs_lin = (lc_span.reshape(_B, 8, 128, 2)
               .transpose(0, 1, 3, 2)
               .reshape(_B * 2 * _N))
    ci_lin = ct_ind.reshape(2, 8, 8, 128).transpose(0, 2, 1, 3)
    cm_lin = ct_mask.reshape(2, 8, 8, 128).transpose(0, 2, 1, 3)
    partials = _sc_loss(
        coord.reshape(_B * 2 * _HW),
        span.reshape(_B * 2 * _HW),
        lc_lin,
        ci_lin,
        lcs_lin,
        cm_lin,
    )
    sums = partials.reshape(_NW, 3, _L).sum(axis=(0, 2))
    mask_sum = sums[2]
    coord_loss = sums[0] / (4.0 * mask_sum + _EPS)
    span_loss = sums[1] / (2.0 * mask_sum + _EPS)
    return (coord_loss, jnp.array(0.0, dtype=jnp.float32), span_loss)


# speedup vs baseline: 7.3514x; 1.0315x over previous
import functools

import jax
import jax.numpy as jnp
from jax import lax
from jax.experimental import pallas as pl
from jax.experimental.pallas import tpu as pltpu
from jax.experimental.pallas import tpu_sc as plsc

_EPS = 1e-4

_B, _N, _H, _W = 16, 1024, 128, 128
_HW = _H * _W
_NC, _NS, _L = 2, 16, 16
_NW = _NC * _NS
_KC = _N * 4

_mesh = plsc.VectorSubcoreMesh(core_axis_name="c", subcore_axis_name="s")


@functools.partial(
    pl.kernel,
    mesh=_mesh,
    out_type=jax.ShapeDtypeStruct((_NW * 48,), jnp.float32),
    compiler_params=pltpu.CompilerParams(needs_layout_passes=False),
    scratch_types=[
        pltpu.VMEM((_HW,), jnp.float32),
        pltpu.VMEM((_HW,), jnp.float32),
        pltpu.VMEM((_KC,), jnp.int32),
        pltpu.VMEM((_KC,), jnp.int32),
        pltpu.VMEM((8, 128), jnp.int32),
        pltpu.VMEM((2 * _N,), jnp.float32),
        pltpu.VMEM((8, 128), jnp.float32),
        pltpu.VMEM((48,), jnp.float32),
        pltpu.SemaphoreType.DMA,
    ],
)
def _sc_loss(coord_hbm, span_hbm, lc_hbm, ci_hbm, lcs_hbm, cm_hbm,
             out_hbm, coord_v, span_v, idx_v, gt_v, ci_v, lcs_v, cm_v,
             stage_v, sem):
    b = lax.axis_index("s")
    c = lax.axis_index("c")
    wid = b * _NC + c
    bt = lax.shift_right_logical(b, 3)
    bs = b & 7

    cps = []
    cps.append(pltpu.async_copy(
        coord_hbm.at[pl.ds((b * 2 + c) * _HW, _HW)], coord_v, sem))
    cps.append(pltpu.async_copy(
        span_hbm.at[pl.ds((b * 2 + c) * _HW, _HW)], span_v, sem))
    cps.append(pltpu.async_copy(
        lc_hbm.at[pl.ds(b * 3 * _KC, _KC)], idx_v, sem))
    cps.append(pltpu.async_copy(
        lc_hbm.at[pl.ds((b * 3 + 1 + c) * _KC, _KC)], gt_v, sem))
    cps.append(pltpu.async_copy(
        lcs_hbm.at[pl.ds(b * 2 * _N, 2 * _N)], lcs_v, sem))
    cps.append(pltpu.async_copy(ci_hbm.at[bt, :, bs], ci_v, sem))
    cps.append(pltpu.async_copy(cm_hbm.at[bt, :, bs], cm_v, sem))
    for cp in cps:
        cp.wait()

    @plsc.parallel_loop(0, _KC // _L, carry=jnp.zeros((_L,), jnp.float32),
                        unroll=4)
    def acc_c(g, acc):
        p0 = g * _L
        idx = idx_v[pl.ds(p0, _L)]
        gt = gt_v[pl.ds(p0, _L)].astype(jnp.float32)
        pred = plsc.load_gather(coord_v, [idx])
        m = cm_v[lax.shift_right_logical(g, 5), pl.ds((g & 7) * _L, _L)]
        return acc + jnp.abs(pred * m - gt * m)

    msel = jnp.where(c == 0, 1.0, 0.0).astype(jnp.float32)

    @plsc.parallel_loop(
        0, _N // _L,
        carry=(jnp.zeros((_L,), jnp.float32), jnp.zeros((_L,), jnp.float32)),
        unroll=4)
    def span_accs(g, carry):
        acc_s, acc_m = carry
        nt = lax.shift_right_logical(g, 3)
        nl0 = (g & 7) * _L
        ind = ci_v[nt, pl.ds(nl0, _L)]
        sp = plsc.load_gather(span_v, [ind])
        gt = lcs_v[pl.ds(nt * 256 + c * 128 + nl0, _L)]
        m = cm_v[nt, pl.ds(nl0, _L)]
        return acc_s + jnp.abs(sp * m - gt * m), acc_m + m * msel

    acc_s, acc_m = span_accs

    stage_v[pl.ds(0, _L)] = acc_c
    stage_v[pl.ds(_L, _L)] = acc_s
    stage_v[pl.ds(2 * _L, _L)] = acc_m
    pltpu.sync_copy(stage_v, out_hbm.at[pl.ds(wid * 48, 48)])


def kernel(coord, span, lc_coords, lc_span, ct_ind, ct_mask):
    lc_lin = (lc_coords.reshape(_B, 8, 128, 4, 3)
              .transpose(0, 4, 1, 3, 2)
              .reshape(_B * 3 * _KC))
    lcs_lin = (lc_span.reshape(_B, 8, 128, 2)
               .transpose(0, 1, 3, 2)
               .reshape(_B * 2 * _N))
    ci_lin = ct_ind.reshape(2, 8, 8, 128).transpose(0, 2, 1, 3)
    cm_lin = ct_mask.reshape(2, 8, 8, 128).transpose(0, 2, 1, 3)
    partials = _sc_loss(
        coord.reshape(_B * 2 * _HW),
        span.reshape(_B * 2 * _HW),
        lc_lin,
        ci_lin,
        lcs_lin,
        cm_lin,
    )
    sums = partials.reshape(_NW, 3, _L).sum(axis=(0, 2))
    mask_sum = sums[2]
    coord_loss = sums[0] / (4.0 * mask_sum + _EPS)
    span_loss = sums[1] / (2.0 * mask_sum + _EPS)
    return (coord_loss, jnp.array(0.0, dtype=jnp.float32), span_loss)
